# baseline (speedup 1.0000x reference)
import numpy as np
import jax
import jax.numpy as jnp
from jax import lax
from jax.experimental import pallas as pl
from jax.experimental.pallas import tpu as pltpu

B = 2
L = 2048
D_MODEL = 768
N_HEADS = 12
HD = D_MODEL // N_HEADS
D_FF = 2048
MARK = 4
U = 24
UP = 32
BH = B * N_HEADS
LB = 128
RB = 512
NLB = L // LB
PRED = 1024


def _np_pos_embed(length, d_model):
    pos = np.arange(length, dtype=np.float64)[:, None]
    div = np.exp(np.arange(0, d_model, 2, dtype=np.float64) * -(np.log(10000.0) / d_model))
    pe = np.zeros((length, d_model), dtype=np.float32)
    pe[:, 0::2] = np.sin(pos * div)
    pe[:, 1::2] = np.cos(pos * div)
    return pe


_POS = _np_pos_embed(L, D_MODEL)


def _sample_counts(fold):
    k = jax.random.fold_in(jax.random.key(42), fold)
    idx = np.asarray(jax.random.randint(k, (L, U), 0, L))
    c = np.zeros((L, L), np.float32)
    np.add.at(c, (np.arange(L)[:, None], idx), 1.0)
    d = np.where(c > 0, np.float32(0.0), np.float32(-1e30))
    return c, d


def _build_counts():
    folds = (0, 1, 100, 101)
    try:
        try:
            cpu = jax.local_devices(backend="cpu")[0]
            with jax.default_device(cpu):
                return {f: _sample_counts(f) for f in folds}
        except Exception:
            return {f: _sample_counts(f) for f in folds}
    except Exception:
        return None


_COUNTS = _build_counts()


def _counts_for(fold):
    if _COUNTS is not None:
        c, d = _COUNTS[fold]
        return jnp.asarray(c), jnp.asarray(d)
    k = jax.random.fold_in(jax.random.key(42), fold)
    idx = jax.random.randint(k, (L, U), 0, L)
    c = jnp.zeros((L, L), jnp.float32).at[jnp.arange(L)[:, None], idx].add(1.0)
    return c, jnp.where(c > 0, 0.0, -1e30).astype(jnp.float32)


def _mmt(x, w):
    return lax.dot_general(x, w, (((1,), (1,)), ((), ())),
                           preferred_element_type=jnp.float32)


def _heads_kernel(x_ref, *refs):
    n = (len(refs) - 1) // 2
    o_ref = refs[-1]
    x = x_ref[0]
    for i in range(n):
        o_ref[0, :, i * D_MODEL:(i + 1) * D_MODEL] = (
            _mmt(x, refs[i][...]) + refs[n + i][...])


def _heads_mm(x, ws, bs):
    n = len(ws)
    wspec = pl.BlockSpec((D_MODEL, D_MODEL), lambda i, j: (0, 0))
    bspec = pl.BlockSpec((1, D_MODEL), lambda i, j: (0, 0))
    return pl.pallas_call(
        _heads_kernel,
        grid=(B, L // RB),
        compiler_params=pltpu.CompilerParams(
            dimension_semantics=("parallel", "parallel")),
        in_specs=[pl.BlockSpec((1, RB, D_MODEL), lambda i, j: (i, j, 0))]
                 + [wspec] * n + [bspec] * n,
        out_specs=pl.BlockSpec((1, RB, n * D_MODEL), lambda i, j: (i, j, 0)),
        out_shape=jax.ShapeDtypeStruct((B, L, n * D_MODEL), jnp.float32),
    )(x, *ws, *[b[None, :] for b in bs])


def _embed_kernel(x_ref, w_ref, p_ref, o_ref):
    o_ref[0] = (jnp.dot(x_ref[0], w_ref[...], preferred_element_type=jnp.float32)
                + p_ref[...])


def _embed(xf, w, pos):
    K = xf.shape[-1]
    return pl.pallas_call(
        _embed_kernel,
        grid=(B, L // RB),
        compiler_params=pltpu.CompilerParams(
            dimension_semantics=("parallel", "parallel")),
        in_specs=[
            pl.BlockSpec((1, RB, K), lambda i, j: (i, j, 0)),
            pl.BlockSpec((K, D_MODEL), lambda i, j: (0, 0)),
            pl.BlockSpec((RB, D_MODEL), lambda i, j: (j, 0)),
        ],
        out_specs=pl.BlockSpec((1, RB, D_MODEL), lambda i, j: (i, j, 0)),
        out_shape=jax.ShapeDtypeStruct((B, L, D_MODEL), jnp.float32),
    )(xf, w, pos)


def _ln_rows(y, g, b):
    m = jnp.mean(y, axis=1, keepdims=True)
    v = jnp.mean((y - m) ** 2, axis=1, keepdims=True)
    return (y - m) / jnp.sqrt(v + 1e-5) * g + b


def _out_ln_kernel(x_ref, w_ref, b_ref, r_ref, g_ref, bb_ref, o_ref):
    y = _mmt(x_ref[0], w_ref[...]) + b_ref[...] + r_ref[0]
    o_ref[0] = _ln_rows(y, g_ref[...], bb_ref[...])


def _out_ln(x, w, b, res, g, beta):
    return pl.pallas_call(
        _out_ln_kernel,
        grid=(B, L // RB),
        compiler_params=pltpu.CompilerParams(
            dimension_semantics=("parallel", "parallel")),
        in_specs=[
            pl.BlockSpec((1, RB, D_MODEL), lambda i, j: (i, j, 0)),
            pl.BlockSpec((D_MODEL, D_MODEL), lambda i, j: (0, 0)),
            pl.BlockSpec((1, D_MODEL), lambda i, j: (0, 0)),
            pl.BlockSpec((1, RB, D_MODEL), lambda i, j: (i, j, 0)),
            pl.BlockSpec((1, D_MODEL), lambda i, j: (0, 0)),
            pl.BlockSpec((1, D_MODEL), lambda i, j: (0, 0)),
        ],
        out_specs=pl.BlockSpec((1, RB, D_MODEL), lambda i, j: (i, j, 0)),
        out_shape=jax.ShapeDtypeStruct((B, L, D_MODEL), jnp.float32),
    )(x, w, b, res, g, beta)


def _gelu(h):
    return 0.5 * h * (1.0 + lax.erf(h * np.float32(1.0 / np.sqrt(2.0))))


def _make_ffn_kernel(second_ln):
    def _ffn_kernel(x_ref, w1_ref, b1_ref, w2_ref, b2_ref, g_ref, bb_ref,
                    *rest):
        o_ref = rest[-1]
        x = x_ref[0]
        h = _gelu(_mmt(x, w1_ref[...]) + b1_ref[...])
        y = _mmt(h, w2_ref[...]) + b2_ref[...] + x
        y = _ln_rows(y, g_ref[...], bb_ref[...])
        if second_ln:
            g2_ref, bb2_ref = rest[0], rest[1]
            y = _ln_rows(y, g2_ref[...], bb2_ref[...])
        o_ref[0] = y
    return _ffn_kernel


def _ffn_ln(x, w1, b1, w2, b2, g, beta, g2=None, beta2=None):
    vec = pl.BlockSpec((1, D_MODEL), lambda i, j: (0, 0))
    vf = pl.BlockSpec((1, D_FF), lambda i, j: (0, 0))
    specs = [
        pl.BlockSpec((1, RB, D_MODEL), lambda i, j: (i, j, 0)),
        pl.BlockSpec((D_FF, D_MODEL), lambda i, j: (0, 0)),
        vf,
        pl.BlockSpec((D_MODEL, D_FF), lambda i, j: (0, 0)),
        vec, vec, vec,
    ]
    args = [x, w1, b1, w2, b2, g, beta]
    if g2 is not None:
        specs += [vec, vec]
        args += [g2, beta2]
    return pl.pallas_call(
        _make_ffn_kernel(g2 is not None),
        grid=(B, L // RB),
        compiler_params=pltpu.CompilerParams(
            dimension_semantics=("parallel", "parallel")),
        in_specs=specs,
        out_specs=pl.BlockSpec((1, RB, D_MODEL), lambda i, j: (i, j, 0)),
        out_shape=jax.ShapeDtypeStruct((B, L, D_MODEL), jnp.float32),
    )(*args)


MB_ = 1024


def _m_stats_kernel(q_ref, k_ref, c_ref, d_ref, m_ref):
    j = pl.program_id(1)
    q = q_ref[0, 0]
    s = lax.dot_general(q, k_ref[0, 0], (((1,), (1,)), ((), ())),
                        preferred_element_type=jnp.float32)
    mx = jnp.max(s + d_ref[pl.ds(j * MB_, MB_), :], axis=1)
    ck = jnp.dot(c_ref[pl.ds(j * MB_, MB_), :], k_ref[0, 0],
                 preferred_element_type=jnp.float32)
    sw = jnp.sum(q * ck, axis=1)
    m_ref[0, 0, :] = mx - sw * np.float32(1.0 / L)


def _m_stats(qa, ka, counts):
    c, d = counts
    return pl.pallas_call(
        _m_stats_kernel,
        grid=(BH, L // MB_),
        compiler_params=pltpu.CompilerParams(
            dimension_semantics=("parallel", "parallel")),
        in_specs=[
            pl.BlockSpec((1, 1, MB_, HD), lambda i, j: (i // N_HEADS, i % N_HEADS, j, 0)),
            pl.BlockSpec((1, 1, L, HD), lambda i, j: (i // N_HEADS, i % N_HEADS, 0, 0)),
            pl.BlockSpec((L, L), lambda i, j: (0, 0)),
            pl.BlockSpec((L, L), lambda i, j: (0, 0)),
        ],
        out_specs=pl.BlockSpec((1, 1, MB_), lambda i, j: (i, 0, j)),
        out_shape=jax.ShapeDtypeStruct((BH, 1, L), jnp.float32),
    )(qa, ka, c, d)


def _topk_kernel(m_ref, t_ref):
    work = m_ref[...].reshape(BH, L)
    coli = lax.broadcasted_iota(jnp.int32, (BH, L), 1)
    for u in range(U):
        mx = jnp.max(work, axis=1, keepdims=True)
        sel = jnp.min(jnp.where(work == mx, coli, L), axis=1, keepdims=True)
        work = jnp.where(coli == sel, -jnp.inf, work)
        t_ref[:, u, :] = jnp.broadcast_to(sel, (BH, 128))
    for u in range(U, UP):
        t_ref[:, u, :] = jnp.full((BH, 128), L, jnp.int32)


def _topk(m):
    return pl.pallas_call(
        _topk_kernel,
        grid=(1,),
        in_specs=[pl.BlockSpec((BH, 1, L), lambda i: (0, 0, 0))],
        out_specs=pl.BlockSpec((BH, UP, 128), lambda i: (0, 0, 0)),
        out_shape=jax.ShapeDtypeStruct((BH, UP, 128), jnp.int32),
    )(m)


def _make_attend_kernel(masked):
    def _attend_kernel(t_ref, q_ref, k_ref, v_ref, o_ref):
        thr = t_ref[0, :, 0:1]
        coli = lax.broadcasted_iota(jnp.int32, (UP, L), 1)
        p = (coli == thr).astype(jnp.float32)
        q = q_ref[0, 0]
        k = k_ref[0, 0]
        v = v_ref[0, 0]
        qr = jnp.dot(p, q, preferred_element_type=jnp.float32)
        s = lax.dot_general(qr, k, (((1,), (1,)), ((), ())),
                            preferred_element_type=jnp.float32) * np.float32(0.125)
        if masked:
            s = jnp.where(coli > thr, -1e30, s)
        s = s - jnp.max(s, axis=1, keepdims=True)
        e = jnp.exp(s)
        attn = e / jnp.sum(e, axis=1, keepdims=True)
        upd = jnp.dot(attn, v, preferred_element_type=jnp.float32)
        ufull = lax.dot_general(p, upd, (((0,), (0,)), ((), ())),
                                preferred_element_type=jnp.float32)
        keep = 1.0 - lax.dot_general(p, jnp.ones((UP, 1), jnp.float32),
                                     (((0,), (0,)), ((), ())),
                                     preferred_element_type=jnp.float32)
        if masked:
            tri = (lax.broadcasted_iota(jnp.int32, (LB, LB), 0)
                   >= lax.broadcasted_iota(jnp.int32, (LB, LB), 1)).astype(jnp.float32)
            carry = jnp.zeros((1, HD), jnp.float32)
            for blk in range(NLB):
                vb = v_ref[0, 0, pl.ds(blk * LB, LB), :]
                cs = jnp.dot(tri, vb, preferred_element_type=jnp.float32) + carry
                o_ref[0, 0, pl.ds(blk * LB, LB), :] = (
                    cs * keep[blk * LB:(blk + 1) * LB] +
                    ufull[blk * LB:(blk + 1) * LB])
                carry = cs[LB - 1:LB, :]
        else:
            mean = jnp.sum(v, axis=0, keepdims=True) * np.float32(1.0 / L)
            o_ref[0, 0] = jnp.broadcast_to(mean, (L, HD)) * keep + ufull
    return _attend_kernel


def _attend(m, qa, ka, va, masked):
    t = _topk(m)
    hb = pl.BlockSpec((1, 1, L, HD), lambda i: (i // N_HEADS, i % N_HEADS, 0, 0))
    return pl.pallas_call(
        _make_attend_kernel(masked),
        grid=(BH,),
        compiler_params=pltpu.CompilerParams(
            dimension_semantics=("parallel",)),
        in_specs=[
            pl.BlockSpec((1, UP, 128), lambda i: (i, 0, 0)),
            hb, hb, hb,
        ],
        out_specs=hb,
        out_shape=jax.ShapeDtypeStruct((B, N_HEADS, L, HD), jnp.float32),
    )(t, qa, ka, va)


def _proj_kernel(x_ref, w_ref, b_ref, o_ref):
    o_ref[0] = _mmt(x_ref[0], w_ref[...]) + b_ref[...]


def _proj(x, w, b):
    return pl.pallas_call(
        _proj_kernel,
        grid=(B, PRED // RB),
        compiler_params=pltpu.CompilerParams(
            dimension_semantics=("parallel", "parallel")),
        in_specs=[
            pl.BlockSpec((1, RB, D_MODEL), lambda i, j: (i, j, 0)),
            pl.BlockSpec((128, D_MODEL), lambda i, j: (0, 0)),
            pl.BlockSpec((1, 128), lambda i, j: (0, 0)),
        ],
        out_specs=pl.BlockSpec((1, RB, 128), lambda i, j: (i, j, 0)),
        out_shape=jax.ShapeDtypeStruct((B, PRED, 128), jnp.float32),
    )(x, w, b)


def _embed_inputs(x, mark, conv_w, temp_w, pos):
    feats = jnp.concatenate(
        [jnp.roll(x, 1, axis=1), x, jnp.roll(x, -1, axis=1), mark], axis=-1)
    nf = feats.shape[-1]
    feats = jnp.pad(feats, ((0, 0), (0, 0), (0, 32 - nf)))
    wf = jnp.concatenate(
        [conv_w.transpose(2, 1, 0).reshape(3 * x.shape[-1], D_MODEL),
         temp_w.T], axis=0)
    wf = jnp.pad(wf, ((0, 32 - nf), (0, 0)))
    return _embed(feats, wf, pos)


def _split_heads(x):
    n = x.shape[-1] // HD
    return x.reshape(B, L, n, HD).transpose(0, 2, 1, 3)


def _merge_heads(x):
    return x.transpose(0, 2, 1, 3).reshape(B, L, D_MODEL)


def _prob_attn(q, k, v, counts, masked):
    m = _m_stats(q, k, counts)
    return _merge_heads(_attend(m, q, k, v, masked))


def kernel(x_enc, x_mark_enc, x_dec, x_mark_dec, params):
    pos = jnp.asarray(_POS)
    counts = {f: _counts_for(f) for f in (0, 1, 100, 101)}

    x = _embed_inputs(x_enc, x_mark_enc, params["enc_emb_conv"],
                      params["enc_emb_temp"], pos)
    for i, p in enumerate(params["enc_layers"]):
        ap = p["attn"]
        qkv = _split_heads(_heads_mm(x, (ap["Wq"], ap["Wk"], ap["Wv"]),
                                     (ap["bq"], ap["bk"], ap["bv"])))
        ctx = _prob_attn(qkv[:, :N_HEADS], qkv[:, N_HEADS:2 * N_HEADS],
                         qkv[:, 2 * N_HEADS:], counts[i], False)
        x = _out_ln(ctx, ap["Wo"], ap["bo"][None, :], x,
                    p["ln1_g"][None, :], p["ln1_b"][None, :])
        last = i == len(params["enc_layers"]) - 1
        x = _ffn_ln(x, p["conv1_w"], p["conv1_b"][None, :],
                    p["conv2_w"], p["conv2_b"][None, :],
                    p["ln2_g"][None, :], p["ln2_b"][None, :],
                    params["enc_norm_g"][None, :] if last else None,
                    params["enc_norm_b"][None, :] if last else None)
    enc = x

    x = _embed_inputs(x_dec, x_mark_dec, params["dec_emb_conv"],
                      params["dec_emb_temp"], pos)
    for i, p in enumerate(params["dec_layers"]):
        sp = p["self_attn"]
        qkv = _split_heads(_heads_mm(x, (sp["Wq"], sp["Wk"], sp["Wv"]),
                                     (sp["bq"], sp["bk"], sp["bv"])))
        ctx = _prob_attn(qkv[:, :N_HEADS], qkv[:, N_HEADS:2 * N_HEADS],
                         qkv[:, 2 * N_HEADS:], counts[100 + 2 * i], True)
        x = _out_ln(ctx, sp["Wo"], sp["bo"][None, :], x,
                    p["ln1_g"][None, :], p["ln1_b"][None, :])
        cp = p["cross_attn"]
        qc = _split_heads(_heads_mm(x, (cp["Wq"],), (cp["bq"],)))
        kvc = _split_heads(_heads_mm(enc, (cp["Wk"], cp["Wv"]),
                                     (cp["bk"], cp["bv"])))
        ctx = _prob_attn(qc, kvc[:, :N_HEADS], kvc[:, N_HEADS:],
                         counts[101 + 2 * i], False)
        x = _out_ln(ctx, cp["Wo"], cp["bo"][None, :], x,
                    p["ln2_g"][None, :], p["ln2_b"][None, :])
        last = i == len(params["dec_layers"]) - 1
        x = _ffn_ln(x, p["conv1_w"], p["conv1_b"][None, :],
                    p["conv2_w"], p["conv2_b"][None, :],
                    p["ln3_g"][None, :], p["ln3_b"][None, :],
                    params["dec_norm_g"][None, :] if last else None,
                    params["dec_norm_b"][None, :] if last else None)

    wp = jnp.pad(params["proj_w"], ((0, 128 - params["proj_w"].shape[0]), (0, 0)))
    bp = jnp.pad(params["proj_b"], (0, 128 - params["proj_b"].shape[0]))[None, :]
    out = _proj(x[:, -PRED:], wp, bp)
    return out[:, :, :params["proj_w"].shape[0]]


# speedup vs baseline: 1.0940x; 1.0940x over previous
import numpy as np
import jax
import jax.numpy as jnp
from jax import lax
from jax.experimental import pallas as pl
from jax.experimental.pallas import tpu as pltpu

B = 2
L = 2048
D_MODEL = 768
N_HEADS = 12
HD = D_MODEL // N_HEADS
D_FF = 2048
MARK = 4
U = 24
UP = 32
BH = B * N_HEADS
LB = 128
RB = 512
NLB = L // LB
PRED = 1024


def _np_pos_embed(length, d_model):
    pos = np.arange(length, dtype=np.float64)[:, None]
    div = np.exp(np.arange(0, d_model, 2, dtype=np.float64) * -(np.log(10000.0) / d_model))
    pe = np.zeros((length, d_model), dtype=np.float32)
    pe[:, 0::2] = np.sin(pos * div)
    pe[:, 1::2] = np.cos(pos * div)
    return pe


_POS = _np_pos_embed(L, D_MODEL)


def _sample_counts(fold):
    k = jax.random.fold_in(jax.random.key(42), fold)
    idx = np.asarray(jax.random.randint(k, (L, U), 0, L))
    c = np.zeros((L, L), np.float32)
    np.add.at(c, (np.arange(L)[:, None], idx), 1.0)
    d = np.where(c > 0, np.float32(0.0), np.float32(-1e30))
    return c, d


def _build_counts():
    folds = (0, 1, 100, 101)
    try:
        try:
            cpu = jax.local_devices(backend="cpu")[0]
            with jax.default_device(cpu):
                return {f: _sample_counts(f) for f in folds}
        except Exception:
            return {f: _sample_counts(f) for f in folds}
    except Exception:
        return None


_COUNTS = _build_counts()


def _counts_for(fold):
    if _COUNTS is not None:
        c, d = _COUNTS[fold]
        return jnp.asarray(c), jnp.asarray(d)
    k = jax.random.fold_in(jax.random.key(42), fold)
    idx = jax.random.randint(k, (L, U), 0, L)
    c = jnp.zeros((L, L), jnp.float32).at[jnp.arange(L)[:, None], idx].add(1.0)
    return c, jnp.where(c > 0, 0.0, -1e30).astype(jnp.float32)


def _mmt(x, w):
    return lax.dot_general(x, w, (((1,), (1,)), ((), ())),
                           preferred_element_type=jnp.float32)


def _heads_kernel(x_ref, *refs):
    n = (len(refs) - 1) // 2
    o_ref = refs[-1]
    x = x_ref[0]
    for i in range(n):
        o_ref[0, :, i * D_MODEL:(i + 1) * D_MODEL] = (
            _mmt(x, refs[i][...]) + refs[n + i][...])


def _heads_mm(x, ws, bs):
    n = len(ws)
    wspec = pl.BlockSpec((D_MODEL, D_MODEL), lambda i, j: (0, 0))
    bspec = pl.BlockSpec((1, D_MODEL), lambda i, j: (0, 0))
    return pl.pallas_call(
        _heads_kernel,
        grid=(B, L // RB),
        compiler_params=pltpu.CompilerParams(
            dimension_semantics=("parallel", "parallel")),
        in_specs=[pl.BlockSpec((1, RB, D_MODEL), lambda i, j: (i, j, 0))]
                 + [wspec] * n + [bspec] * n,
        out_specs=pl.BlockSpec((1, RB, n * D_MODEL), lambda i, j: (i, j, 0)),
        out_shape=jax.ShapeDtypeStruct((B, L, n * D_MODEL), jnp.float32),
    )(x, *ws, *[b[None, :] for b in bs])


def _embed_kernel(x_ref, w_ref, p_ref, o_ref):
    o_ref[0] = (jnp.dot(x_ref[0], w_ref[...], preferred_element_type=jnp.float32)
                + p_ref[...])


def _embed(xf, w, pos):
    K = xf.shape[-1]
    return pl.pallas_call(
        _embed_kernel,
        grid=(B, L // RB),
        compiler_params=pltpu.CompilerParams(
            dimension_semantics=("parallel", "parallel")),
        in_specs=[
            pl.BlockSpec((1, RB, K), lambda i, j: (i, j, 0)),
            pl.BlockSpec((K, D_MODEL), lambda i, j: (0, 0)),
            pl.BlockSpec((RB, D_MODEL), lambda i, j: (j, 0)),
        ],
        out_specs=pl.BlockSpec((1, RB, D_MODEL), lambda i, j: (i, j, 0)),
        out_shape=jax.ShapeDtypeStruct((B, L, D_MODEL), jnp.float32),
    )(xf, w, pos)


def _ln_rows(y, g, b):
    m = jnp.mean(y, axis=1, keepdims=True)
    v = jnp.mean((y - m) ** 2, axis=1, keepdims=True)
    return (y - m) / jnp.sqrt(v + 1e-5) * g + b


def _out_ln_kernel(x_ref, w_ref, b_ref, r_ref, g_ref, bb_ref, o_ref):
    y = _mmt(x_ref[0], w_ref[...]) + b_ref[...] + r_ref[0]
    o_ref[0] = _ln_rows(y, g_ref[...], bb_ref[...])


def _out_ln(x, w, b, res, g, beta):
    return pl.pallas_call(
        _out_ln_kernel,
        grid=(B, L // RB),
        compiler_params=pltpu.CompilerParams(
            dimension_semantics=("parallel", "parallel")),
        in_specs=[
            pl.BlockSpec((1, RB, D_MODEL), lambda i, j: (i, j, 0)),
            pl.BlockSpec((D_MODEL, D_MODEL), lambda i, j: (0, 0)),
            pl.BlockSpec((1, D_MODEL), lambda i, j: (0, 0)),
            pl.BlockSpec((1, RB, D_MODEL), lambda i, j: (i, j, 0)),
            pl.BlockSpec((1, D_MODEL), lambda i, j: (0, 0)),
            pl.BlockSpec((1, D_MODEL), lambda i, j: (0, 0)),
        ],
        out_specs=pl.BlockSpec((1, RB, D_MODEL), lambda i, j: (i, j, 0)),
        out_shape=jax.ShapeDtypeStruct((B, L, D_MODEL), jnp.float32),
    )(x, w, b, res, g, beta)


def _gelu(h):
    return 0.5 * h * (1.0 + lax.erf(h * np.float32(1.0 / np.sqrt(2.0))))


def _make_ffn_kernel(second_ln):
    def _ffn_kernel(x_ref, w1_ref, b1_ref, w2_ref, b2_ref, g_ref, bb_ref,
                    *rest):
        o_ref = rest[-1]
        x = x_ref[0]
        h = _gelu(_mmt(x, w1_ref[...]) + b1_ref[...])
        y = _mmt(h, w2_ref[...]) + b2_ref[...] + x
        y = _ln_rows(y, g_ref[...], bb_ref[...])
        if second_ln:
            g2_ref, bb2_ref = rest[0], rest[1]
            y = _ln_rows(y, g2_ref[...], bb2_ref[...])
        o_ref[0] = y
    return _ffn_kernel


def _ffn_ln(x, w1, b1, w2, b2, g, beta, g2=None, beta2=None):
    vec = pl.BlockSpec((1, D_MODEL), lambda i, j: (0, 0))
    vf = pl.BlockSpec((1, D_FF), lambda i, j: (0, 0))
    specs = [
        pl.BlockSpec((1, RB, D_MODEL), lambda i, j: (i, j, 0)),
        pl.BlockSpec((D_FF, D_MODEL), lambda i, j: (0, 0)),
        vf,
        pl.BlockSpec((D_MODEL, D_FF), lambda i, j: (0, 0)),
        vec, vec, vec,
    ]
    args = [x, w1, b1, w2, b2, g, beta]
    if g2 is not None:
        specs += [vec, vec]
        args += [g2, beta2]
    return pl.pallas_call(
        _make_ffn_kernel(g2 is not None),
        grid=(B, L // RB),
        compiler_params=pltpu.CompilerParams(
            dimension_semantics=("parallel", "parallel")),
        in_specs=specs,
        out_specs=pl.BlockSpec((1, RB, D_MODEL), lambda i, j: (i, j, 0)),
        out_shape=jax.ShapeDtypeStruct((B, L, D_MODEL), jnp.float32),
    )(*args)


MB_ = 1024


def _m_stats_kernel(q_ref, k_ref, c_ref, d_ref, m_ref):
    j = pl.program_id(1)
    q = q_ref[0, 0]
    s = lax.dot_general(q, k_ref[0, 0], (((1,), (1,)), ((), ())),
                        preferred_element_type=jnp.float32)
    mx = jnp.max(s + d_ref[pl.ds(j * MB_, MB_), :], axis=1)
    sw = jnp.sum(s * c_ref[pl.ds(j * MB_, MB_), :], axis=1)
    m_ref[0, 0, :] = mx - sw * np.float32(1.0 / L)


def _m_stats(qa, ka, counts):
    c, d = counts
    return pl.pallas_call(
        _m_stats_kernel,
        grid=(BH, L // MB_),
        compiler_params=pltpu.CompilerParams(
            dimension_semantics=("parallel", "parallel")),
        in_specs=[
            pl.BlockSpec((1, 1, MB_, HD), lambda i, j: (i // N_HEADS, i % N_HEADS, j, 0)),
            pl.BlockSpec((1, 1, L, HD), lambda i, j: (i // N_HEADS, i % N_HEADS, 0, 0)),
            pl.BlockSpec((L, L), lambda i, j: (0, 0)),
            pl.BlockSpec((L, L), lambda i, j: (0, 0)),
        ],
        out_specs=pl.BlockSpec((1, 1, MB_), lambda i, j: (i, 0, j)),
        out_shape=jax.ShapeDtypeStruct((BH, 1, L), jnp.float32),
    )(qa, ka, c, d)


def _topk_kernel(m_ref, t_ref):
    work = m_ref[...].reshape(BH, L)
    coli = lax.broadcasted_iota(jnp.int32, (BH, L), 1)
    for u in range(U):
        mx = jnp.max(work, axis=1, keepdims=True)
        sel = jnp.min(jnp.where(work == mx, coli, L), axis=1, keepdims=True)
        work = jnp.where(coli == sel, -jnp.inf, work)
        t_ref[:, u, :] = jnp.broadcast_to(sel, (BH, 128))
    for u in range(U, UP):
        t_ref[:, u, :] = jnp.full((BH, 128), L, jnp.int32)


def _topk(m):
    return pl.pallas_call(
        _topk_kernel,
        grid=(1,),
        in_specs=[pl.BlockSpec((BH, 1, L), lambda i: (0, 0, 0))],
        out_specs=pl.BlockSpec((BH, UP, 128), lambda i: (0, 0, 0)),
        out_shape=jax.ShapeDtypeStruct((BH, UP, 128), jnp.int32),
    )(m)


def _make_attend_kernel(masked):
    def _attend_kernel(t_ref, q_ref, k_ref, v_ref, o_ref):
        thr = t_ref[0, :, 0:1]
        coli = lax.broadcasted_iota(jnp.int32, (UP, L), 1)
        p = (coli == thr).astype(jnp.float32)
        q = q_ref[0, 0]
        k = k_ref[0, 0]
        v = v_ref[0, 0]
        qr = jnp.dot(p, q, preferred_element_type=jnp.float32)
        s = lax.dot_general(qr, k, (((1,), (1,)), ((), ())),
                            preferred_element_type=jnp.float32) * np.float32(0.125)
        if masked:
            s = jnp.where(coli > thr, -1e30, s)
        s = s - jnp.max(s, axis=1, keepdims=True)
        e = jnp.exp(s)
        attn = e / jnp.sum(e, axis=1, keepdims=True)
        upd = jnp.dot(attn, v, preferred_element_type=jnp.float32)
        ufull = lax.dot_general(p, upd, (((0,), (0,)), ((), ())),
                                preferred_element_type=jnp.float32)
        keep = 1.0 - lax.dot_general(p, jnp.ones((UP, 1), jnp.float32),
                                     (((0,), (0,)), ((), ())),
                                     preferred_element_type=jnp.float32)
        if masked:
            tri = (lax.broadcasted_iota(jnp.int32, (LB, LB), 0)
                   >= lax.broadcasted_iota(jnp.int32, (LB, LB), 1)).astype(jnp.float32)
            carry = jnp.zeros((1, HD), jnp.float32)
            for blk in range(NLB):
                vb = v_ref[0, 0, pl.ds(blk * LB, LB), :]
                cs = jnp.dot(tri, vb, preferred_element_type=jnp.float32) + carry
                o_ref[0, 0, pl.ds(blk * LB, LB), :] = (
                    cs * keep[blk * LB:(blk + 1) * LB] +
                    ufull[blk * LB:(blk + 1) * LB])
                carry = cs[LB - 1:LB, :]
        else:
            mean = jnp.sum(v, axis=0, keepdims=True) * np.float32(1.0 / L)
            o_ref[0, 0] = jnp.broadcast_to(mean, (L, HD)) * keep + ufull
    return _attend_kernel


def _attend(m, qa, ka, va, masked):
    t = _topk(m)
    hb = pl.BlockSpec((1, 1, L, HD), lambda i: (i // N_HEADS, i % N_HEADS, 0, 0))
    return pl.pallas_call(
        _make_attend_kernel(masked),
        grid=(BH,),
        compiler_params=pltpu.CompilerParams(
            dimension_semantics=("parallel",)),
        in_specs=[
            pl.BlockSpec((1, UP, 128), lambda i: (i, 0, 0)),
            hb, hb, hb,
        ],
        out_specs=hb,
        out_shape=jax.ShapeDtypeStruct((B, N_HEADS, L, HD), jnp.float32),
    )(t, qa, ka, va)


def _proj_kernel(x_ref, w_ref, b_ref, o_ref):
    o_ref[0] = _mmt(x_ref[0], w_ref[...]) + b_ref[...]


def _proj(x, w, b):
    return pl.pallas_call(
        _proj_kernel,
        grid=(B, PRED // RB),
        compiler_params=pltpu.CompilerParams(
            dimension_semantics=("parallel", "parallel")),
        in_specs=[
            pl.BlockSpec((1, RB, D_MODEL), lambda i, j: (i, j, 0)),
            pl.BlockSpec((128, D_MODEL), lambda i, j: (0, 0)),
            pl.BlockSpec((1, 128), lambda i, j: (0, 0)),
        ],
        out_specs=pl.BlockSpec((1, RB, 128), lambda i, j: (i, j, 0)),
        out_shape=jax.ShapeDtypeStruct((B, PRED, 128), jnp.float32),
    )(x, w, b)


def _embed_inputs(x, mark, conv_w, temp_w, pos):
    feats = jnp.concatenate(
        [jnp.roll(x, 1, axis=1), x, jnp.roll(x, -1, axis=1), mark], axis=-1)
    nf = feats.shape[-1]
    feats = jnp.pad(feats, ((0, 0), (0, 0), (0, 32 - nf)))
    wf = jnp.concatenate(
        [conv_w.transpose(2, 1, 0).reshape(3 * x.shape[-1], D_MODEL),
         temp_w.T], axis=0)
    wf = jnp.pad(wf, ((0, 32 - nf), (0, 0)))
    return _embed(feats, wf, pos)


def _split_heads(x):
    n = x.shape[-1] // HD
    return x.reshape(B, L, n, HD).transpose(0, 2, 1, 3)


def _merge_heads(x):
    return x.transpose(0, 2, 1, 3).reshape(B, L, D_MODEL)


def _prob_attn(q, k, v, counts, masked):
    m = _m_stats(q, k, counts)
    return _merge_heads(_attend(m, q, k, v, masked))


def kernel(x_enc, x_mark_enc, x_dec, x_mark_dec, params):
    pos = jnp.asarray(_POS)
    counts = {f: _counts_for(f) for f in (0, 1, 100, 101)}

    x = _embed_inputs(x_enc, x_mark_enc, params["enc_emb_conv"],
                      params["enc_emb_temp"], pos)
    for i, p in enumerate(params["enc_layers"]):
        ap = p["attn"]
        qkv = _split_heads(_heads_mm(x, (ap["Wq"], ap["Wk"], ap["Wv"]),
                                     (ap["bq"], ap["bk"], ap["bv"])))
        ctx = _prob_attn(qkv[:, :N_HEADS], qkv[:, N_HEADS:2 * N_HEADS],
                         qkv[:, 2 * N_HEADS:], counts[i], False)
        x = _out_ln(ctx, ap["Wo"], ap["bo"][None, :], x,
                    p["ln1_g"][None, :], p["ln1_b"][None, :])
        last = i == len(params["enc_layers"]) - 1
        x = _ffn_ln(x, p["conv1_w"], p["conv1_b"][None, :],
                    p["conv2_w"], p["conv2_b"][None, :],
                    p["ln2_g"][None, :], p["ln2_b"][None, :],
                    params["enc_norm_g"][None, :] if last else None,
                    params["enc_norm_b"][None, :] if last else None)
    enc = x

    x = _embed_inputs(x_dec, x_mark_dec, params["dec_emb_conv"],
                      params["dec_emb_temp"], pos)
    for i, p in enumerate(params["dec_layers"]):
        sp = p["self_attn"]
        qkv = _split_heads(_heads_mm(x, (sp["Wq"], sp["Wk"], sp["Wv"]),
                                     (sp["bq"], sp["bk"], sp["bv"])))
        ctx = _prob_attn(qkv[:, :N_HEADS], qkv[:, N_HEADS:2 * N_HEADS],
                         qkv[:, 2 * N_HEADS:], counts[100 + 2 * i], True)
        x = _out_ln(ctx, sp["Wo"], sp["bo"][None, :], x,
                    p["ln1_g"][None, :], p["ln1_b"][None, :])
        cp = p["cross_attn"]
        qc = _split_heads(_heads_mm(x, (cp["Wq"],), (cp["bq"],)))
        kvc = _split_heads(_heads_mm(enc, (cp["Wk"], cp["Wv"]),
                                     (cp["bk"], cp["bv"])))
        ctx = _prob_attn(qc, kvc[:, :N_HEADS], kvc[:, N_HEADS:],
                         counts[101 + 2 * i], False)
        x = _out_ln(ctx, cp["Wo"], cp["bo"][None, :], x,
                    p["ln2_g"][None, :], p["ln2_b"][None, :])
        last = i == len(params["dec_layers"]) - 1
        x = _ffn_ln(x, p["conv1_w"], p["conv1_b"][None, :],
                    p["conv2_w"], p["conv2_b"][None, :],
                    p["ln3_g"][None, :], p["ln3_b"][None, :],
                    params["dec_norm_g"][None, :] if last else None,
                    params["dec_norm_b"][None, :] if last else None)

    wp = jnp.pad(params["proj_w"], ((0, 128 - params["proj_w"].shape[0]), (0, 0)))
    bp = jnp.pad(params["proj_b"], (0, 128 - params["proj_b"].shape[0]))[None, :]
    out = _proj(x[:, -PRED:], wp, bp)
    return out[:, :, :params["proj_w"].shape[0]]
